# Initial kernel scaffold; baseline (speedup 1.0000x reference)
import jax
import jax.numpy as jnp
from jax.experimental import pallas as pl


def kernel(x, turn, tiles, zeros_param):
    raise NotImplementedError("write your pallas kernel here")


# speedup vs baseline: 1467.6002x; 1467.6002x over previous
import functools

import jax
import jax.numpy as jnp
from jax import lax
from jax.experimental import pallas as pl
from jax.experimental.pallas import tpu as pltpu
from jax.experimental.pallas import tpu_sc as plsc

_K = 36
_V = 768
_NW = 32
_R = 128


def _sc_counts(xflat, B):
    rows_per_w = B // _NW
    n_chunks = rows_per_w // _R
    mesh = plsc.VectorSubcoreMesh(core_axis_name="c", subcore_axis_name="s")

    @functools.partial(
        pl.kernel,
        mesh=mesh,
        out_type=jax.ShapeDtypeStruct((B * _V,), jnp.float32),
        compiler_params=pltpu.CompilerParams(needs_layout_passes=False),
        scratch_types=[
            pltpu.VMEM((_R * _K,), jnp.int32),
            pltpu.VMEM((_R * _V,), jnp.float32),
        ],
    )
    def k(x_hbm, c_hbm, x_v, c_v):
        wid = lax.axis_index("s") * 2 + lax.axis_index("c")
        lanes = lax.iota(jnp.int32, 16)
        ones = jnp.full((16,), 1.0, jnp.float32)
        zeros = jnp.zeros((16,), jnp.float32)

        def zbody(j, carry):
            for u in range(8):
                c_v[pl.ds((j * 8 + u) * 16, 16)] = zeros
            return carry

        lax.fori_loop(0, _R * _V // (16 * 8), zbody, 0)

        def scatter_pass(sign):
            vals = ones * sign

            def gbody(g, carry):
                rows = g * 16 + lanes
                xbase = rows * _K
                cbase = rows * _V
                for i in range(_K):
                    idx = plsc.load_gather(x_v, [xbase + i])
                    valid = idx < _V
                    plsc.addupdate_scatter(c_v, [cbase + idx], vals, mask=valid)
                return carry

            lax.fori_loop(0, _R // 16, gbody, 0)

        def chunk_body(t, carry):
            row0 = wid * rows_per_w + t * _R
            pltpu.sync_copy(x_hbm.at[pl.ds(row0 * _K, _R * _K)], x_v)
            scatter_pass(1.0)
            pltpu.sync_copy(c_v, c_hbm.at[pl.ds(row0 * _V, _R * _V)])
            scatter_pass(-1.0)
            return carry

        lax.fori_loop(0, n_chunks, chunk_body, 0)

    return k(xflat)


def _tc_reduce(C, T_aug, turn, B):
    BM = 512

    def body(c_ref, t_ref, u_ref, o_ref):
        Cb = c_ref[...]
        P = jnp.dot(Cb, t_ref[...], preferred_element_type=jnp.float32)
        q = jnp.sum(Cb * P[:, :_V], axis=1, keepdims=True)
        n = jnp.sum(Cb, axis=1, keepdims=True)
        c3 = Cb[:, 3:4]
        c443 = Cb[:, 443:444]
        p3 = P[:, 3:4]
        p443 = P[:, 443:444]
        p768 = P[:, _V:_V + 1]
        p769 = P[:, _V + 1:_V + 2]
        t33 = t_ref[3, 3]
        t443 = t_ref[443, 443]
        t440_3 = t_ref[440, 3]
        outw = q - c3 * (p3 + p768) + c3 * c3 * t33
        outb = (q - c443 * (p443 + p769) + c443 * c443 * t443
                + (float(_K * _K) - n * n) * t440_3)
        o_ref[...] = jnp.where(u_ref[...] == 1, outw, outb)

    return pl.pallas_call(
        body,
        grid=(B // BM,),
        in_specs=[
            pl.BlockSpec((BM, _V), lambda i: (i, 0)),
            pl.BlockSpec((_V, _V + 128), lambda i: (0, 0)),
            pl.BlockSpec((BM, 1), lambda i: (i, 0)),
        ],
        out_specs=pl.BlockSpec((BM, 1), lambda i: (i, 0)),
        out_shape=jax.ShapeDtypeStruct((B, 1), jnp.float32),
    )(C, T_aug, turn)


def kernel(x, turn, tiles, zeros_param):
    B = x.shape[0]
    x32 = x.astype(jnp.int32).reshape(B * _K)
    C = _sc_counts(x32, B).reshape(B, _V)

    T2 = tiles.reshape(_V, _V)
    T_aug = jnp.concatenate(
        [T2, T2[3:4, :].T, T2[443:444, :].T,
         jnp.zeros((_V, 126), jnp.float32)], axis=1)

    out = _tc_reduce(C, T_aug, turn.astype(jnp.int32), B)
    return (out, jnp.zeros((1,), dtype=out.dtype))
